# Initial kernel scaffold; baseline (speedup 1.0000x reference)
import jax
import jax.numpy as jnp
from jax.experimental import pallas as pl


def kernel(ids, edge_index, nid, text_emb, deepwalk_emb, W_text, b_text, W_dw, b_dw, ln_scale, ln_bias, W_self, W_neigh, b_conv):
    raise NotImplementedError("write your pallas kernel here")


# speedup vs baseline: 3.5222x; 3.5222x over previous
import functools

import jax
import jax.numpy as jnp
from jax import lax
from jax.experimental import pallas as pl
from jax.experimental.pallas import tpu as pltpu
from jax.experimental.pallas import tpu_sc as plsc

N = 10000
E = 320000
B = 1024
D = 128
DT = 768

NC = 2
NS = 16
CHUNK = 80
EPW = E // NS
NCHUNK = EPW // CHUNK
NPAD = 10240
ROWS_PW = NPAD // NS
DEGW = 128

_MB = 10
MBLK = N // 5


def _mesh():
    return plsc.VectorSubcoreMesh(core_axis_name="c", subcore_axis_name="s")


def _agg_body(g0, g1, e0, e1, zeros, out0, out1, acc, gi, si, buf, sem):
    c = lax.axis_index("c")
    s = lax.axis_index("s")
    r0 = s * ROWS_PW
    pltpu.sync_copy(zeros.at[pl.ds(r0, ROWS_PW)], acc.at[pl.ds(r0, ROWS_PW)])
    plsc.subcore_barrier()

    def run_dir(g_ref, eg_ref, es_ref):
        base = s * EPW

        def body(i, carry):
            off = pl.multiple_of(base + i * CHUNK, 8)
            pltpu.sync_copy(eg_ref.at[pl.ds(off, CHUNK)], gi)
            pltpu.sync_copy(es_ref.at[pl.ds(off, CHUNK)], si)
            pltpu.async_copy(g_ref.at[gi], buf, sem).wait()
            pltpu.sync_copy(buf, acc.at[si], add=True)
            return carry

        lax.fori_loop(0, NCHUNK, body, 0)

    @pl.when(c == 0)
    def _():
        run_dir(g0, e0, e1)

    @pl.when(c == 1)
    def _():
        run_dir(g1, e1, e0)

    plsc.subcore_barrier()

    @pl.when(c == 0)
    def _():
        pltpu.sync_copy(acc.at[pl.ds(r0, ROWS_PW)], out0.at[pl.ds(r0, ROWS_PW)])

    @pl.when(c == 1)
    def _():
        pltpu.sync_copy(acc.at[pl.ds(r0, ROWS_PW)], out1.at[pl.ds(r0, ROWS_PW)])


@jax.jit
def _agg(g0, g1, e0, e1, zeros):
    return pl.kernel(
        _agg_body,
        out_type=(
            jax.ShapeDtypeStruct((NPAD, D), jnp.float32),
            jax.ShapeDtypeStruct((NPAD, D), jnp.float32),
        ),
        mesh=_mesh(),
        scratch_types=[
            pltpu.VMEM_SHARED((NPAD, D), jnp.float32),
            pltpu.VMEM((CHUNK,), jnp.int32),
            pltpu.VMEM((CHUNK,), jnp.int32),
            pltpu.VMEM((CHUNK, D), jnp.float32),
            pltpu.SemaphoreType.DMA,
        ],
    )(g0, g1, e0, e1, zeros)


def _deg_body(e0, e1, zeros16, ones16, out0, out1, acc, si, ones_v, sem):
    c = lax.axis_index("c")
    s = lax.axis_index("s")
    r0 = s * ROWS_PW
    pltpu.sync_copy(zeros16.at[pl.ds(r0, ROWS_PW)], acc.at[pl.ds(r0, ROWS_PW)])
    pltpu.sync_copy(ones16, ones_v)
    plsc.subcore_barrier()

    def run_dir(es_ref):
        base = s * EPW

        def body(i, carry):
            off = pl.multiple_of(base + i * CHUNK, 8)
            pltpu.sync_copy(es_ref.at[pl.ds(off, CHUNK)], si)
            pltpu.sync_copy(ones_v, acc.at[si], add=True)
            return carry

        lax.fori_loop(0, NCHUNK, body, 0)

    @pl.when(c == 0)
    def _():
        run_dir(e1)

    @pl.when(c == 1)
    def _():
        run_dir(e0)

    plsc.subcore_barrier()

    @pl.when(c == 0)
    def _():
        pltpu.sync_copy(acc.at[pl.ds(r0, ROWS_PW)], out0.at[pl.ds(r0, ROWS_PW)])

    @pl.when(c == 1)
    def _():
        pltpu.sync_copy(acc.at[pl.ds(r0, ROWS_PW)], out1.at[pl.ds(r0, ROWS_PW)])


@jax.jit
def _deg(e0, e1, zeros16, ones16):
    return pl.kernel(
        _deg_body,
        out_type=(
            jax.ShapeDtypeStruct((NPAD, DEGW), jnp.float32),
            jax.ShapeDtypeStruct((NPAD, DEGW), jnp.float32),
        ),
        mesh=_mesh(),
        scratch_types=[
            pltpu.VMEM_SHARED((NPAD, DEGW), jnp.float32),
            pltpu.VMEM((CHUNK,), jnp.int32),
            pltpu.VMEM((CHUNK, DEGW), jnp.float32),
            pltpu.SemaphoreType.DMA,
        ],
    )(e0, e1, zeros16, ones16)


BPW = B // (NC * NS)


def _gather_body(table, idx, out, idx_v, rows_v, sem):
    wid = lax.axis_index("s") * NC + lax.axis_index("c")
    base = wid * BPW
    pltpu.sync_copy(idx.at[pl.ds(base, BPW)], idx_v)
    pltpu.async_copy(table.at[idx_v], rows_v, sem).wait()
    pltpu.sync_copy(rows_v, out.at[pl.ds(base, BPW)])


@jax.jit
def _gather(table, idx):
    return pl.kernel(
        _gather_body,
        out_type=jax.ShapeDtypeStruct((B, D), jnp.float32),
        mesh=_mesh(),
        scratch_types=[
            pltpu.VMEM((BPW,), jnp.int32),
            pltpu.VMEM((BPW, D), jnp.float32),
            pltpu.SemaphoreType.DMA,
        ],
    )(table, idx)


def _prologue_body(t_ref, wt_ref, bt_ref, d_ref, wd_ref, bd_ref, o_ref):
    acc = jnp.dot(t_ref[...], wt_ref[...], preferred_element_type=jnp.float32)
    acc += jnp.dot(d_ref[...], wd_ref[...], preferred_element_type=jnp.float32)
    o_ref[...] = acc + bt_ref[...] + bd_ref[...]


@jax.jit
def _prologue(text_emb, WtT, bt, dw_emb, WdT, bd):
    return pl.pallas_call(
        _prologue_body,
        grid=(5,),
        in_specs=[
            pl.BlockSpec((MBLK, DT), lambda i: (i, 0)),
            pl.BlockSpec((DT, D), lambda i: (0, 0)),
            pl.BlockSpec((1, D), lambda i: (0, 0)),
            pl.BlockSpec((MBLK, D), lambda i: (i, 0)),
            pl.BlockSpec((D, D), lambda i: (0, 0)),
            pl.BlockSpec((1, D), lambda i: (0, 0)),
        ],
        out_specs=pl.BlockSpec((MBLK, D), lambda i: (i, 0)),
        out_shape=jax.ShapeDtypeStruct((N, D), jnp.float32),
    )(text_emb, WtT, bt, dw_emb, WdT, bd)


def _ln(x, g, b):
    mu = jnp.mean(x, axis=-1, keepdims=True)
    var = jnp.mean((x - mu) ** 2, axis=-1, keepdims=True)
    return (x - mu) / jnp.sqrt(var + 1e-5) * g + b


def _lnmm_body(f_ref, g0s, g0b, g1s, g1b, ws0, wn0, ws1, wn1,
               z0_ref, q0_ref, z1_ref, q1_ref):
    f = f_ref[...]
    hn0 = _ln(f, g0s[...], g0b[...])
    hn1 = _ln(f, g1s[...], g1b[...])
    z0_ref[...] = jnp.dot(hn0, ws0[...], preferred_element_type=jnp.float32)
    q0_ref[...] = jnp.dot(hn0, wn0[...], preferred_element_type=jnp.float32)
    z1_ref[...] = jnp.dot(hn1, ws1[...], preferred_element_type=jnp.float32)
    q1_ref[...] = jnp.dot(hn1, wn1[...], preferred_element_type=jnp.float32)


@jax.jit
def _lnmm(feats, g0s, g0b, g1s, g1b, ws0, wn0, ws1, wn1):
    vec = pl.BlockSpec((1, D), lambda i: (0, 0))
    mat = pl.BlockSpec((D, D), lambda i: (0, 0))
    blk = pl.BlockSpec((MBLK, D), lambda i: (i, 0))
    return pl.pallas_call(
        _lnmm_body,
        grid=(5,),
        in_specs=[blk, vec, vec, vec, vec, mat, mat, mat, mat],
        out_specs=[blk, blk, blk, blk],
        out_shape=[jax.ShapeDtypeStruct((N, D), jnp.float32)] * 4,
    )(feats, g0s, g0b, g1s, g1b, ws0, wn0, ws1, wn1)


def _combine_body(f_ref, z0_ref, s0_ref, d0_ref, z1_ref, s1_ref, d1_ref,
                  b0_ref, b1_ref, o_ref):
    d0 = jnp.maximum(d0_ref[...][:, 0:1], 1.0)
    d1 = jnp.maximum(d1_ref[...][:, 0:1], 1.0)
    y0 = jax.nn.relu(z0_ref[...] + s0_ref[...] / d0 + b0_ref[...])
    y1 = jax.nn.relu(z1_ref[...] + s1_ref[...] / d1 + b1_ref[...])
    o_ref[...] = f_ref[...] + y0 + y1


@jax.jit
def _combine(feats, z0, s0, d0, z1, s1, d1, b0, b1):
    blk = pl.BlockSpec((MBLK, D), lambda i: (i, 0))
    dblk = pl.BlockSpec((MBLK, DEGW), lambda i: (i, 0))
    vec = pl.BlockSpec((1, D), lambda i: (0, 0))
    return pl.pallas_call(
        _combine_body,
        grid=(5,),
        in_specs=[blk, blk, blk, dblk, blk, blk, dblk, vec, vec],
        out_specs=blk,
        out_shape=jax.ShapeDtypeStruct((N, D), jnp.float32),
    )(feats, z0, s0, d0, z1, s1, d1, b0, b1)


def _norm_body(x_ref, o_ref):
    x = x_ref[...]
    o_ref[...] = x * lax.rsqrt(jnp.sum(x * x, axis=-1, keepdims=True))


@jax.jit
def _normalize(rows):
    return pl.pallas_call(
        _norm_body,
        out_shape=jax.ShapeDtypeStruct((B, D), jnp.float32),
    )(rows)


def kernel(ids, edge_index, nid, text_emb, deepwalk_emb, W_text, b_text,
           W_dw, b_dw, ln_scale, ln_bias, W_self, W_neigh, b_conv):
    e0 = edge_index[0]
    e1 = edge_index[1]
    zeros = jnp.zeros((NPAD, D), jnp.float32)
    zeros16 = jnp.zeros((NPAD, DEGW), jnp.float32)
    ones16 = jnp.ones((CHUNK, DEGW), jnp.float32)

    feats = _prologue(text_emb, W_text.T, b_text.reshape(1, D),
                      deepwalk_emb, W_dw.T, b_dw.reshape(1, D))
    d0, d1 = _deg(e0, e1, zeros16, ones16)
    for l in range(2):
        z0, g0, z1, g1 = _lnmm(
            feats,
            ln_scale[l, 0].reshape(1, D), ln_bias[l, 0].reshape(1, D),
            ln_scale[l, 1].reshape(1, D), ln_bias[l, 1].reshape(1, D),
            W_self[l, 0].T, W_neigh[l, 0].T,
            W_self[l, 1].T, W_neigh[l, 1].T,
        )
        s0, s1 = _agg(g0, g1, e0, e1, zeros)
        feats = _combine(feats, z0, s0, d0, z1, s1, d1,
                         b_conv[l, 0].reshape(1, D), b_conv[l, 1].reshape(1, D))
    rows = _gather(feats, ids)
    return _normalize(rows)


# speedup vs baseline: 5.7843x; 1.6423x over previous
import functools

import jax
import jax.numpy as jnp
from jax import lax
from jax.experimental import pallas as pl
from jax.experimental.pallas import tpu as pltpu
from jax.experimental.pallas import tpu_sc as plsc

N = 10000
E = 320000
B = 1024
D = 128
DT = 768

NC = 2
NS = 16
CHUNK = 80
EPW = E // NS
NCHUNK = EPW // CHUNK
NPAIR = NCHUNK // 2
NPAD = 10240
ROWS_PW = NPAD // NS
DEGW = 128

_MB = 10
MBLK = N // 5


def _mesh():
    return plsc.VectorSubcoreMesh(core_axis_name="c", subcore_axis_name="s")


def _agg_body(g0, g1, e0, e1, zeros, out0, out1, acc,
              gi0, si0, gi1, si1, buf0, buf1, sg0, sg1):
    c = lax.axis_index("c")
    s = lax.axis_index("s")
    r0 = s * ROWS_PW
    pltpu.sync_copy(zeros.at[pl.ds(r0, ROWS_PW)], acc.at[pl.ds(r0, ROWS_PW)])
    plsc.subcore_barrier()

    def run_dir(g_ref, eg_ref, es_ref):
        base = s * EPW

        def load_idx(ch, gi, si):
            off = pl.multiple_of(base + ch * CHUNK, 8)
            pltpu.sync_copy(eg_ref.at[pl.ds(off, CHUNK)], gi)
            pltpu.sync_copy(es_ref.at[pl.ds(off, CHUNK)], si)

        load_idx(0, gi0, si0)
        pltpu.async_copy(g_ref.at[gi0], buf0, sg0)

        def body(j, carry):
            load_idx(2 * j + 1, gi1, si1)
            pltpu.async_copy(g_ref.at[gi1], buf1, sg1)
            pltpu.make_async_copy(g_ref.at[gi0], buf0, sg0).wait()
            pltpu.sync_copy(buf0, acc.at[si0], add=True)

            @pl.when(j < NPAIR - 1)
            def _():
                load_idx(2 * j + 2, gi0, si0)
                pltpu.async_copy(g_ref.at[gi0], buf0, sg0)

            pltpu.make_async_copy(g_ref.at[gi1], buf1, sg1).wait()
            pltpu.sync_copy(buf1, acc.at[si1], add=True)
            return carry

        lax.fori_loop(0, NPAIR, body, 0)

    @pl.when(c == 0)
    def _():
        run_dir(g0, e0, e1)

    @pl.when(c == 1)
    def _():
        run_dir(g1, e1, e0)

    plsc.subcore_barrier()

    @pl.when(c == 0)
    def _():
        pltpu.sync_copy(acc.at[pl.ds(r0, ROWS_PW)], out0.at[pl.ds(r0, ROWS_PW)])

    @pl.when(c == 1)
    def _():
        pltpu.sync_copy(acc.at[pl.ds(r0, ROWS_PW)], out1.at[pl.ds(r0, ROWS_PW)])


@jax.jit
def _agg(g0, g1, e0, e1, zeros):
    return pl.kernel(
        _agg_body,
        out_type=(
            jax.ShapeDtypeStruct((NPAD, D), jnp.float32),
            jax.ShapeDtypeStruct((NPAD, D), jnp.float32),
        ),
        mesh=_mesh(),
        scratch_types=[
            pltpu.VMEM_SHARED((NPAD, D), jnp.float32),
            pltpu.VMEM((CHUNK,), jnp.int32),
            pltpu.VMEM((CHUNK,), jnp.int32),
            pltpu.VMEM((CHUNK,), jnp.int32),
            pltpu.VMEM((CHUNK,), jnp.int32),
            pltpu.VMEM((CHUNK, D), jnp.float32),
            pltpu.VMEM((CHUNK, D), jnp.float32),
            pltpu.SemaphoreType.DMA,
            pltpu.SemaphoreType.DMA,
        ],
    )(g0, g1, e0, e1, zeros)


def _deg_body(e0, e1, zeros16, ones16, out0, out1, acc, si0, si1, ones_v, sd):
    c = lax.axis_index("c")
    s = lax.axis_index("s")
    r0 = s * ROWS_PW
    pltpu.sync_copy(zeros16.at[pl.ds(r0, ROWS_PW)], acc.at[pl.ds(r0, ROWS_PW)])
    pltpu.sync_copy(ones16, ones_v)
    plsc.subcore_barrier()

    def run_dir(es_ref):
        base = s * EPW

        def load_idx(ch, si):
            off = pl.multiple_of(base + ch * CHUNK, 8)
            pltpu.sync_copy(es_ref.at[pl.ds(off, CHUNK)], si)

        load_idx(0, si0)
        pltpu.async_copy(ones_v, acc.at[si0], sd, add=True)

        def body(j, carry):
            load_idx(2 * j + 1, si1)
            pltpu.async_copy(ones_v, acc.at[si1], sd, add=True)
            pltpu.make_async_copy(ones_v, acc.at[si0], sd).wait()

            @pl.when(j < NPAIR - 1)
            def _():
                load_idx(2 * j + 2, si0)
                pltpu.async_copy(ones_v, acc.at[si0], sd, add=True)

            pltpu.make_async_copy(ones_v, acc.at[si1], sd).wait()
            return carry

        lax.fori_loop(0, NPAIR, body, 0)

    @pl.when(c == 0)
    def _():
        run_dir(e1)

    @pl.when(c == 1)
    def _():
        run_dir(e0)

    plsc.subcore_barrier()

    @pl.when(c == 0)
    def _():
        pltpu.sync_copy(acc.at[pl.ds(r0, ROWS_PW)], out0.at[pl.ds(r0, ROWS_PW)])

    @pl.when(c == 1)
    def _():
        pltpu.sync_copy(acc.at[pl.ds(r0, ROWS_PW)], out1.at[pl.ds(r0, ROWS_PW)])


@jax.jit
def _deg(e0, e1, zeros16, ones16):
    return pl.kernel(
        _deg_body,
        out_type=(
            jax.ShapeDtypeStruct((NPAD, DEGW), jnp.float32),
            jax.ShapeDtypeStruct((NPAD, DEGW), jnp.float32),
        ),
        mesh=_mesh(),
        scratch_types=[
            pltpu.VMEM_SHARED((NPAD, DEGW), jnp.float32),
            pltpu.VMEM((CHUNK,), jnp.int32),
            pltpu.VMEM((CHUNK,), jnp.int32),
            pltpu.VMEM((CHUNK, DEGW), jnp.float32),
            pltpu.SemaphoreType.DMA,
        ],
    )(e0, e1, zeros16, ones16)


BPW = B // (NC * NS)


def _gather_body(table, idx, out, idx_v, rows_v, sem):
    wid = lax.axis_index("s") * NC + lax.axis_index("c")
    base = wid * BPW
    pltpu.sync_copy(idx.at[pl.ds(base, BPW)], idx_v)
    pltpu.async_copy(table.at[idx_v], rows_v, sem).wait()
    pltpu.sync_copy(rows_v, out.at[pl.ds(base, BPW)])


@jax.jit
def _gather(table, idx):
    return pl.kernel(
        _gather_body,
        out_type=jax.ShapeDtypeStruct((B, D), jnp.float32),
        mesh=_mesh(),
        scratch_types=[
            pltpu.VMEM((BPW,), jnp.int32),
            pltpu.VMEM((BPW, D), jnp.float32),
            pltpu.SemaphoreType.DMA,
        ],
    )(table, idx)


def _prologue_body(t_ref, wt_ref, bt_ref, d_ref, wd_ref, bd_ref, o_ref):
    acc = jnp.dot(t_ref[...], wt_ref[...], preferred_element_type=jnp.float32)
    acc += jnp.dot(d_ref[...], wd_ref[...], preferred_element_type=jnp.float32)
    o_ref[...] = acc + bt_ref[...] + bd_ref[...]


@jax.jit
def _prologue(text_emb, WtT, bt, dw_emb, WdT, bd):
    return pl.pallas_call(
        _prologue_body,
        grid=(5,),
        in_specs=[
            pl.BlockSpec((MBLK, DT), lambda i: (i, 0)),
            pl.BlockSpec((DT, D), lambda i: (0, 0)),
            pl.BlockSpec((1, D), lambda i: (0, 0)),
            pl.BlockSpec((MBLK, D), lambda i: (i, 0)),
            pl.BlockSpec((D, D), lambda i: (0, 0)),
            pl.BlockSpec((1, D), lambda i: (0, 0)),
        ],
        out_specs=pl.BlockSpec((MBLK, D), lambda i: (i, 0)),
        out_shape=jax.ShapeDtypeStruct((N, D), jnp.float32),
    )(text_emb, WtT, bt, dw_emb, WdT, bd)


def _ln(x, g, b):
    mu = jnp.mean(x, axis=-1, keepdims=True)
    var = jnp.mean((x - mu) ** 2, axis=-1, keepdims=True)
    return (x - mu) / jnp.sqrt(var + 1e-5) * g + b


def _lnmm_body(f_ref, g0s, g0b, g1s, g1b, ws0, wn0, ws1, wn1,
               z0_ref, q0_ref, z1_ref, q1_ref):
    f = f_ref[...]
    hn0 = _ln(f, g0s[...], g0b[...])
    hn1 = _ln(f, g1s[...], g1b[...])
    z0_ref[...] = jnp.dot(hn0, ws0[...], preferred_element_type=jnp.float32)
    q0_ref[...] = jnp.dot(hn0, wn0[...], preferred_element_type=jnp.float32)
    z1_ref[...] = jnp.dot(hn1, ws1[...], preferred_element_type=jnp.float32)
    q1_ref[...] = jnp.dot(hn1, wn1[...], preferred_element_type=jnp.float32)


@jax.jit
def _lnmm(feats, g0s, g0b, g1s, g1b, ws0, wn0, ws1, wn1):
    vec = pl.BlockSpec((1, D), lambda i: (0, 0))
    mat = pl.BlockSpec((D, D), lambda i: (0, 0))
    blk = pl.BlockSpec((MBLK, D), lambda i: (i, 0))
    return pl.pallas_call(
        _lnmm_body,
        grid=(5,),
        in_specs=[blk, vec, vec, vec, vec, mat, mat, mat, mat],
        out_specs=[blk, blk, blk, blk],
        out_shape=[jax.ShapeDtypeStruct((N, D), jnp.float32)] * 4,
    )(feats, g0s, g0b, g1s, g1b, ws0, wn0, ws1, wn1)


def _combine_body(f_ref, z0_ref, s0_ref, d0_ref, z1_ref, s1_ref, d1_ref,
                  b0_ref, b1_ref, o_ref):
    d0 = jnp.maximum(d0_ref[...][:, 0:1], 1.0)
    d1 = jnp.maximum(d1_ref[...][:, 0:1], 1.0)
    y0 = jax.nn.relu(z0_ref[...] + s0_ref[...] / d0 + b0_ref[...])
    y1 = jax.nn.relu(z1_ref[...] + s1_ref[...] / d1 + b1_ref[...])
    o_ref[...] = f_ref[...] + y0 + y1


@jax.jit
def _combine(feats, z0, s0, d0, z1, s1, d1, b0, b1):
    blk = pl.BlockSpec((MBLK, D), lambda i: (i, 0))
    dblk = pl.BlockSpec((MBLK, DEGW), lambda i: (i, 0))
    vec = pl.BlockSpec((1, D), lambda i: (0, 0))
    return pl.pallas_call(
        _combine_body,
        grid=(5,),
        in_specs=[blk, blk, blk, dblk, blk, blk, dblk, vec, vec],
        out_specs=blk,
        out_shape=jax.ShapeDtypeStruct((N, D), jnp.float32),
    )(feats, z0, s0, d0, z1, s1, d1, b0, b1)


def _norm_body(x_ref, o_ref):
    x = x_ref[...]
    o_ref[...] = x * lax.rsqrt(jnp.sum(x * x, axis=-1, keepdims=True))


@jax.jit
def _normalize(rows):
    return pl.pallas_call(
        _norm_body,
        out_shape=jax.ShapeDtypeStruct((B, D), jnp.float32),
    )(rows)


def kernel(ids, edge_index, nid, text_emb, deepwalk_emb, W_text, b_text,
           W_dw, b_dw, ln_scale, ln_bias, W_self, W_neigh, b_conv):
    e0 = edge_index[0]
    e1 = edge_index[1]
    zeros = jnp.zeros((NPAD, D), jnp.float32)
    zeros16 = jnp.zeros((NPAD, DEGW), jnp.float32)
    ones16 = jnp.ones((CHUNK, DEGW), jnp.float32)

    feats = _prologue(text_emb, W_text.T, b_text.reshape(1, D),
                      deepwalk_emb, W_dw.T, b_dw.reshape(1, D))
    d0, d1 = _deg(e0, e1, zeros16, ones16)
    for l in range(2):
        z0, g0, z1, g1 = _lnmm(
            feats,
            ln_scale[l, 0].reshape(1, D), ln_bias[l, 0].reshape(1, D),
            ln_scale[l, 1].reshape(1, D), ln_bias[l, 1].reshape(1, D),
            W_self[l, 0].T, W_neigh[l, 0].T,
            W_self[l, 1].T, W_neigh[l, 1].T,
        )
        s0, s1 = _agg(g0, g1, e0, e1, zeros)
        feats = _combine(feats, z0, s0, d0, z1, s1, d1,
                         b_conv[l, 0].reshape(1, D), b_conv[l, 1].reshape(1, D))
    rows = _gather(feats, ids)
    return _normalize(rows)


# speedup vs baseline: 6.3536x; 1.0984x over previous
import jax
import jax.numpy as jnp
from jax import lax
from jax.experimental import pallas as pl
from jax.experimental.pallas import tpu as pltpu
from jax.experimental.pallas import tpu_sc as plsc

N = 10000
E = 320000
B = 1024
D = 128
DT = 768

NC = 2
NS = 16
CH = 100
NB = 8
EPW = E // NS
RPT = EPW // CH
NBATCH = RPT // NB
NPAD = 10240
ROWS_PW = NPAD // NS
DEGW = 128

MBLK = N // 5


def _mesh():
    return plsc.VectorSubcoreMesh(core_axis_name="c", subcore_axis_name="s")


def _agg_body(g0, g1, e0r, e1r, zeros, out0, out1, acc,
              gb, sb, buf0, buf1, sg0, sg1, ss0, ss1):
    c = lax.axis_index("c")
    s = lax.axis_index("s")
    r0 = s * ROWS_PW
    pltpu.sync_copy(zeros.at[pl.ds(r0, ROWS_PW)], acc.at[pl.ds(r0, ROWS_PW)])
    plsc.subcore_barrier()

    def run_dir(g_ref, eg_ref, es_ref):
        row0 = s * RPT

        def batch(b, carry):
            r = pl.multiple_of(row0 + b * NB, 8)
            pltpu.sync_copy(eg_ref.at[pl.ds(r, NB)], gb)
            pltpu.sync_copy(es_ref.at[pl.ds(r, NB)], sb)

            bufs = (buf0, buf1)
            sgs = (sg0, sg1)
            sss = (ss0, ss1)
            pltpu.async_copy(g_ref.at[gb.at[0]], buf0, sg0)
            for j in range(NB):
                bj, sgj, ssj = bufs[j % 2], sgs[j % 2], sss[j % 2]
                pltpu.make_async_copy(g_ref.at[gb.at[j]], bj, sgj).wait()
                pltpu.async_copy(bj, acc.at[sb.at[j]], ssj, add=True)
                if j + 1 < NB:
                    if j >= 1:
                        pltpu.make_async_copy(
                            bufs[(j + 1) % 2], acc.at[sb.at[j - 1]],
                            sss[(j + 1) % 2]).wait()
                    pltpu.async_copy(g_ref.at[gb.at[j + 1]],
                                     bufs[(j + 1) % 2], sgs[(j + 1) % 2])
            pltpu.make_async_copy(buf0, acc.at[sb.at[NB - 2]], ss0).wait()
            pltpu.make_async_copy(buf1, acc.at[sb.at[NB - 1]], ss1).wait()
            return carry

        lax.fori_loop(0, NBATCH, batch, 0)

    @pl.when(c == 0)
    def _():
        run_dir(g0, e0r, e1r)

    @pl.when(c == 1)
    def _():
        run_dir(g1, e1r, e0r)

    plsc.subcore_barrier()

    @pl.when(c == 0)
    def _():
        pltpu.sync_copy(acc.at[pl.ds(r0, ROWS_PW)], out0.at[pl.ds(r0, ROWS_PW)])

    @pl.when(c == 1)
    def _():
        pltpu.sync_copy(acc.at[pl.ds(r0, ROWS_PW)], out1.at[pl.ds(r0, ROWS_PW)])


@jax.jit
def _agg(g0, g1, e0r, e1r, zeros):
    return pl.kernel(
        _agg_body,
        out_type=(
            jax.ShapeDtypeStruct((NPAD, D), jnp.float32),
            jax.ShapeDtypeStruct((NPAD, D), jnp.float32),
        ),
        mesh=_mesh(),
        scratch_types=[
            pltpu.VMEM_SHARED((NPAD, D), jnp.float32),
            pltpu.VMEM((NB, CH), jnp.int32),
            pltpu.VMEM((NB, CH), jnp.int32),
            pltpu.VMEM((CH, D), jnp.float32),
            pltpu.VMEM((CH, D), jnp.float32),
            pltpu.SemaphoreType.DMA,
            pltpu.SemaphoreType.DMA,
            pltpu.SemaphoreType.DMA,
            pltpu.SemaphoreType.DMA,
        ],
    )(g0, g1, e0r, e1r, zeros)


def _deg_body(e0r, e1r, zerosd, onesd, out0, out1, acc, sb, ones_v, sd):
    c = lax.axis_index("c")
    s = lax.axis_index("s")
    r0 = s * ROWS_PW
    pltpu.sync_copy(zerosd.at[pl.ds(r0, ROWS_PW)], acc.at[pl.ds(r0, ROWS_PW)])
    pltpu.sync_copy(onesd, ones_v)
    plsc.subcore_barrier()

    def run_dir(es_ref):
        row0 = s * RPT

        def batch(b, carry):
            r = pl.multiple_of(row0 + b * NB, 8)
            pltpu.sync_copy(es_ref.at[pl.ds(r, NB)], sb)
            for j in range(NB):
                pltpu.async_copy(ones_v, acc.at[sb.at[j]], sd, add=True)
            for j in range(NB):
                pltpu.make_async_copy(ones_v, acc.at[sb.at[j]], sd).wait()
            return carry

        lax.fori_loop(0, NBATCH, batch, 0)

    @pl.when(c == 0)
    def _():
        run_dir(e1r)

    @pl.when(c == 1)
    def _():
        run_dir(e0r)

    plsc.subcore_barrier()

    @pl.when(c == 0)
    def _():
        pltpu.sync_copy(acc.at[pl.ds(r0, ROWS_PW)], out0.at[pl.ds(r0, ROWS_PW)])

    @pl.when(c == 1)
    def _():
        pltpu.sync_copy(acc.at[pl.ds(r0, ROWS_PW)], out1.at[pl.ds(r0, ROWS_PW)])


@jax.jit
def _deg(e0r, e1r, zerosd, onesd):
    return pl.kernel(
        _deg_body,
        out_type=(
            jax.ShapeDtypeStruct((NPAD, DEGW), jnp.float32),
            jax.ShapeDtypeStruct((NPAD, DEGW), jnp.float32),
        ),
        mesh=_mesh(),
        scratch_types=[
            pltpu.VMEM_SHARED((NPAD, DEGW), jnp.float32),
            pltpu.VMEM((NB, CH), jnp.int32),
            pltpu.VMEM((CH, DEGW), jnp.float32),
            pltpu.SemaphoreType.DMA,
        ],
    )(e0r, e1r, zerosd, onesd)


BPW = B // (NC * NS)


def _gather_body(table, idx, out, idx_v, rows_v, sem):
    wid = lax.axis_index("s") * NC + lax.axis_index("c")
    base = wid * BPW
    pltpu.sync_copy(idx.at[pl.ds(base, BPW)], idx_v)
    pltpu.async_copy(table.at[idx_v], rows_v, sem).wait()
    pltpu.sync_copy(rows_v, out.at[pl.ds(base, BPW)])


@jax.jit
def _gather(table, idx):
    return pl.kernel(
        _gather_body,
        out_type=jax.ShapeDtypeStruct((B, D), jnp.float32),
        mesh=_mesh(),
        scratch_types=[
            pltpu.VMEM((BPW,), jnp.int32),
            pltpu.VMEM((BPW, D), jnp.float32),
            pltpu.SemaphoreType.DMA,
        ],
    )(table, idx)


def _prologue_body(t_ref, wt_ref, bt_ref, d_ref, wd_ref, bd_ref, o_ref):
    acc = jnp.dot(t_ref[...], wt_ref[...], preferred_element_type=jnp.float32)
    acc += jnp.dot(d_ref[...], wd_ref[...], preferred_element_type=jnp.float32)
    o_ref[...] = acc + bt_ref[...] + bd_ref[...]


@jax.jit
def _prologue(text_emb, WtT, bt, dw_emb, WdT, bd):
    return pl.pallas_call(
        _prologue_body,
        grid=(5,),
        in_specs=[
            pl.BlockSpec((MBLK, DT), lambda i: (i, 0)),
            pl.BlockSpec((DT, D), lambda i: (0, 0)),
            pl.BlockSpec((1, D), lambda i: (0, 0)),
            pl.BlockSpec((MBLK, D), lambda i: (i, 0)),
            pl.BlockSpec((D, D), lambda i: (0, 0)),
            pl.BlockSpec((1, D), lambda i: (0, 0)),
        ],
        out_specs=pl.BlockSpec((MBLK, D), lambda i: (i, 0)),
        out_shape=jax.ShapeDtypeStruct((N, D), jnp.float32),
    )(text_emb, WtT, bt, dw_emb, WdT, bd)


def _ln(x, g, b):
    mu = jnp.mean(x, axis=-1, keepdims=True)
    var = jnp.mean((x - mu) ** 2, axis=-1, keepdims=True)
    return (x - mu) / jnp.sqrt(var + 1e-5) * g + b


def _lnmm_body(f_ref, g0s, g0b, g1s, g1b, ws0, wn0, ws1, wn1,
               z0_ref, q0_ref, z1_ref, q1_ref):
    f = f_ref[...]
    hn0 = _ln(f, g0s[...], g0b[...])
    hn1 = _ln(f, g1s[...], g1b[...])
    z0_ref[...] = jnp.dot(hn0, ws0[...], preferred_element_type=jnp.float32)
    q0_ref[...] = jnp.dot(hn0, wn0[...], preferred_element_type=jnp.float32)
    z1_ref[...] = jnp.dot(hn1, ws1[...], preferred_element_type=jnp.float32)
    q1_ref[...] = jnp.dot(hn1, wn1[...], preferred_element_type=jnp.float32)


@jax.jit
def _lnmm(feats, g0s, g0b, g1s, g1b, ws0, wn0, ws1, wn1):
    vec = pl.BlockSpec((1, D), lambda i: (0, 0))
    mat = pl.BlockSpec((D, D), lambda i: (0, 0))
    blk = pl.BlockSpec((MBLK, D), lambda i: (i, 0))
    return pl.pallas_call(
        _lnmm_body,
        grid=(5,),
        in_specs=[blk, vec, vec, vec, vec, mat, mat, mat, mat],
        out_specs=[blk, blk, blk, blk],
        out_shape=[jax.ShapeDtypeStruct((N, D), jnp.float32)] * 4,
    )(feats, g0s, g0b, g1s, g1b, ws0, wn0, ws1, wn1)


def _combine_body(f_ref, z0_ref, s0_ref, d0_ref, z1_ref, s1_ref, d1_ref,
                  b0_ref, b1_ref, o_ref):
    d0 = jnp.maximum(d0_ref[...][:, 0:1], 1.0)
    d1 = jnp.maximum(d1_ref[...][:, 0:1], 1.0)
    y0 = jax.nn.relu(z0_ref[...] + s0_ref[...] / d0 + b0_ref[...])
    y1 = jax.nn.relu(z1_ref[...] + s1_ref[...] / d1 + b1_ref[...])
    o_ref[...] = f_ref[...] + y0 + y1


@jax.jit
def _combine(feats, z0, s0, d0, z1, s1, d1, b0, b1):
    blk = pl.BlockSpec((MBLK, D), lambda i: (i, 0))
    dblk = pl.BlockSpec((MBLK, DEGW), lambda i: (i, 0))
    vec = pl.BlockSpec((1, D), lambda i: (0, 0))
    return pl.pallas_call(
        _combine_body,
        grid=(5,),
        in_specs=[blk, blk, blk, dblk, blk, blk, dblk, vec, vec],
        out_specs=blk,
        out_shape=jax.ShapeDtypeStruct((N, D), jnp.float32),
    )(feats, z0, s0, d0, z1, s1, d1, b0, b1)


def _norm_body(x_ref, o_ref):
    x = x_ref[...]
    o_ref[...] = x * lax.rsqrt(jnp.sum(x * x, axis=-1, keepdims=True))


@jax.jit
def _normalize(rows):
    return pl.pallas_call(
        _norm_body,
        out_shape=jax.ShapeDtypeStruct((B, D), jnp.float32),
    )(rows)


def kernel(ids, edge_index, nid, text_emb, deepwalk_emb, W_text, b_text,
           W_dw, b_dw, ln_scale, ln_bias, W_self, W_neigh, b_conv):
    e0r = edge_index[0].reshape(E // CH, CH)
    e1r = edge_index[1].reshape(E // CH, CH)
    zeros = jnp.zeros((NPAD, D), jnp.float32)
    zerosd = jnp.zeros((NPAD, DEGW), jnp.float32)
    onesd = jnp.ones((CH, DEGW), jnp.float32)

    feats = _prologue(text_emb, W_text.T, b_text.reshape(1, D),
                      deepwalk_emb, W_dw.T, b_dw.reshape(1, D))
    d0, d1 = _deg(e0r, e1r, zerosd, onesd)
    for l in range(2):
        z0, g0, z1, g1 = _lnmm(
            feats,
            ln_scale[l, 0].reshape(1, D), ln_bias[l, 0].reshape(1, D),
            ln_scale[l, 1].reshape(1, D), ln_bias[l, 1].reshape(1, D),
            W_self[l, 0].T, W_neigh[l, 0].T,
            W_self[l, 1].T, W_neigh[l, 1].T,
        )
        s0, s1 = _agg(g0, g1, e0r, e1r, zeros)
        feats = _combine(feats, z0, s0, d0, z1, s1, d1,
                         b_conv[l, 0].reshape(1, D), b_conv[l, 1].reshape(1, D))
    rows = _gather(feats, ids)
    return _normalize(rows)


# speedup vs baseline: 6.6093x; 1.0403x over previous
import jax
import jax.numpy as jnp
from jax import lax
from jax.experimental import pallas as pl
from jax.experimental.pallas import tpu as pltpu
from jax.experimental.pallas import tpu_sc as plsc

N = 10000
E = 320000
B = 1024
D = 128
DT = 768

NC = 2
NS = 16
CH = 100
NB = 8
EPW = E // NS
RPT = EPW // CH
NBATCH = RPT // NB
NPAD = 10240
ROWS_PW = NPAD // NS
DEGW = 128

MBLK = N // 5


def _mesh():
    return plsc.VectorSubcoreMesh(core_axis_name="c", subcore_axis_name="s")


def _agg_body(g0, g1, e0r, e1r, zeros, out0, out1, acc,
              gb, sb, buf0, buf1, sg0, sg1, ss0, ss1):
    c = lax.axis_index("c")
    s = lax.axis_index("s")
    r0 = s * ROWS_PW
    pltpu.sync_copy(zeros.at[pl.ds(r0, ROWS_PW)], acc.at[pl.ds(r0, ROWS_PW)])
    plsc.subcore_barrier()

    def run_dir(g_ref, eg_ref, es_ref):
        row0 = s * RPT

        def batch(b, carry):
            r = pl.multiple_of(row0 + b * NB, 8)
            pltpu.sync_copy(eg_ref.at[pl.ds(r, NB)], gb)
            pltpu.sync_copy(es_ref.at[pl.ds(r, NB)], sb)

            bufs = (buf0, buf1)
            sgs = (sg0, sg1)
            sss = (ss0, ss1)
            pltpu.async_copy(g_ref.at[gb.at[0]], buf0, sg0)
            for j in range(NB):
                bj, sgj, ssj = bufs[j % 2], sgs[j % 2], sss[j % 2]
                pltpu.make_async_copy(g_ref.at[gb.at[j]], bj, sgj).wait()
                pltpu.async_copy(bj, acc.at[sb.at[j]], ssj, add=True)
                if j + 1 < NB:
                    if j >= 1:
                        pltpu.make_async_copy(
                            bufs[(j + 1) % 2], acc.at[sb.at[j - 1]],
                            sss[(j + 1) % 2]).wait()
                    pltpu.async_copy(g_ref.at[gb.at[j + 1]],
                                     bufs[(j + 1) % 2], sgs[(j + 1) % 2])
            pltpu.make_async_copy(buf0, acc.at[sb.at[NB - 2]], ss0).wait()
            pltpu.make_async_copy(buf1, acc.at[sb.at[NB - 1]], ss1).wait()
            return carry

        lax.fori_loop(0, NBATCH, batch, 0)

    @pl.when(c == 0)
    def _():
        run_dir(g0, e0r, e1r)

    @pl.when(c == 1)
    def _():
        run_dir(g1, e1r, e0r)

    plsc.subcore_barrier()

    @pl.when(c == 0)
    def _():
        pltpu.sync_copy(acc.at[pl.ds(r0, ROWS_PW)], out0.at[pl.ds(r0, ROWS_PW)])

    @pl.when(c == 1)
    def _():
        pltpu.sync_copy(acc.at[pl.ds(r0, ROWS_PW)], out1.at[pl.ds(r0, ROWS_PW)])


@jax.jit
def _agg(g0, g1, e0r, e1r, zeros):
    return pl.kernel(
        _agg_body,
        out_type=(
            jax.ShapeDtypeStruct((NPAD, D), jnp.float32),
            jax.ShapeDtypeStruct((NPAD, D), jnp.float32),
        ),
        mesh=_mesh(),
        scratch_types=[
            pltpu.VMEM_SHARED((NPAD, D), jnp.float32),
            pltpu.VMEM((NB, CH), jnp.int32),
            pltpu.VMEM((NB, CH), jnp.int32),
            pltpu.VMEM((CH, D), jnp.float32),
            pltpu.VMEM((CH, D), jnp.float32),
            pltpu.SemaphoreType.DMA,
            pltpu.SemaphoreType.DMA,
            pltpu.SemaphoreType.DMA,
            pltpu.SemaphoreType.DMA,
        ],
    )(g0, g1, e0r, e1r, zeros)


def _deg_body(e0r, e1r, zerosd, onesd, out0, out1, acc, sb, ones_v, sd):
    c = lax.axis_index("c")
    s = lax.axis_index("s")
    r0 = s * ROWS_PW
    pltpu.sync_copy(zerosd.at[pl.ds(r0, ROWS_PW)], acc.at[pl.ds(r0, ROWS_PW)])
    pltpu.sync_copy(onesd, ones_v)
    plsc.subcore_barrier()

    def run_dir(es_ref):
        row0 = s * RPT

        def batch(b, carry):
            r = pl.multiple_of(row0 + b * NB, 8)
            pltpu.sync_copy(es_ref.at[pl.ds(r, NB)], sb)
            for j in range(NB):
                pltpu.async_copy(ones_v, acc.at[sb.at[j]], sd, add=True)
            for j in range(NB):
                pltpu.make_async_copy(ones_v, acc.at[sb.at[j]], sd).wait()
            return carry

        lax.fori_loop(0, NBATCH, batch, 0)

    @pl.when(c == 0)
    def _():
        run_dir(e1r)

    @pl.when(c == 1)
    def _():
        run_dir(e0r)

    plsc.subcore_barrier()

    @pl.when(c == 0)
    def _():
        pltpu.sync_copy(acc.at[pl.ds(r0, ROWS_PW)], out0.at[pl.ds(r0, ROWS_PW)])

    @pl.when(c == 1)
    def _():
        pltpu.sync_copy(acc.at[pl.ds(r0, ROWS_PW)], out1.at[pl.ds(r0, ROWS_PW)])


@jax.jit
def _deg(e0r, e1r, zerosd, onesd):
    return pl.kernel(
        _deg_body,
        out_type=(
            jax.ShapeDtypeStruct((NPAD, DEGW), jnp.float32),
            jax.ShapeDtypeStruct((NPAD, DEGW), jnp.float32),
        ),
        mesh=_mesh(),
        scratch_types=[
            pltpu.VMEM_SHARED((NPAD, DEGW), jnp.float32),
            pltpu.VMEM((NB, CH), jnp.int32),
            pltpu.VMEM((CH, DEGW), jnp.float32),
            pltpu.SemaphoreType.DMA,
        ],
    )(e0r, e1r, zerosd, onesd)


BPW = B // (NC * NS)


def _gather_body(table, idx, out, idx_v, rows_v, sem):
    wid = lax.axis_index("s") * NC + lax.axis_index("c")
    base = wid * BPW
    pltpu.sync_copy(idx.at[pl.ds(base, BPW)], idx_v)
    pltpu.async_copy(table.at[idx_v], rows_v, sem).wait()
    pltpu.sync_copy(rows_v, out.at[pl.ds(base, BPW)])


@jax.jit
def _gather(table, idx):
    return pl.kernel(
        _gather_body,
        out_type=jax.ShapeDtypeStruct((B, D), jnp.float32),
        mesh=_mesh(),
        scratch_types=[
            pltpu.VMEM((BPW,), jnp.int32),
            pltpu.VMEM((BPW, D), jnp.float32),
            pltpu.SemaphoreType.DMA,
        ],
    )(table, idx)


def _prologue_body(t_ref, wt_ref, bt_ref, d_ref, wd_ref, bd_ref, o_ref):
    acc = jnp.dot(t_ref[...], wt_ref[...], preferred_element_type=jnp.float32)
    acc += jnp.dot(d_ref[...], wd_ref[...], preferred_element_type=jnp.float32)
    o_ref[...] = acc + bt_ref[...] + bd_ref[...]


@jax.jit
def _prologue(text_emb, WtT, bt, dw_emb, WdT, bd):
    return pl.pallas_call(
        _prologue_body,
        grid=(5,),
        in_specs=[
            pl.BlockSpec((MBLK, DT), lambda i: (i, 0)),
            pl.BlockSpec((DT, D), lambda i: (0, 0)),
            pl.BlockSpec((1, D), lambda i: (0, 0)),
            pl.BlockSpec((MBLK, D), lambda i: (i, 0)),
            pl.BlockSpec((D, D), lambda i: (0, 0)),
            pl.BlockSpec((1, D), lambda i: (0, 0)),
        ],
        out_specs=pl.BlockSpec((MBLK, D), lambda i: (i, 0)),
        out_shape=jax.ShapeDtypeStruct((N, D), jnp.float32),
    )(text_emb, WtT, bt, dw_emb, WdT, bd)


def _ln(x, g, b):
    mu = jnp.mean(x, axis=-1, keepdims=True)
    var = jnp.mean((x - mu) ** 2, axis=-1, keepdims=True)
    return (x - mu) / jnp.sqrt(var + 1e-5) * g + b


def _lnmm_body(f_ref, g0s, g0b, g1s, g1b, ws0, wn0, ws1, wn1,
               z0_ref, q0_ref, z1_ref, q1_ref):
    f = f_ref[...]
    hn0 = _ln(f, g0s[...], g0b[...])
    hn1 = _ln(f, g1s[...], g1b[...])
    z0_ref[...] = jnp.dot(hn0, ws0[...], preferred_element_type=jnp.float32)
    q0_ref[...] = jnp.dot(hn0, wn0[...], preferred_element_type=jnp.float32)
    z1_ref[...] = jnp.dot(hn1, ws1[...], preferred_element_type=jnp.float32)
    q1_ref[...] = jnp.dot(hn1, wn1[...], preferred_element_type=jnp.float32)


@jax.jit
def _lnmm(feats, g0s, g0b, g1s, g1b, ws0, wn0, ws1, wn1):
    vec = pl.BlockSpec((1, D), lambda i: (0, 0))
    mat = pl.BlockSpec((D, D), lambda i: (0, 0))
    blk = pl.BlockSpec((MBLK, D), lambda i: (i, 0))
    return pl.pallas_call(
        _lnmm_body,
        grid=(5,),
        in_specs=[blk, vec, vec, vec, vec, mat, mat, mat, mat],
        out_specs=[blk, blk, blk, blk],
        out_shape=[jax.ShapeDtypeStruct((N, D), jnp.float32)] * 4,
    )(feats, g0s, g0b, g1s, g1b, ws0, wn0, ws1, wn1)


def _pro_lnmm_body(t_ref, wt_ref, bt_ref, d_ref, wd_ref, bd_ref,
                   g0s, g0b, g1s, g1b, ws0, wn0, ws1, wn1,
                   f_ref, z0_ref, q0_ref, z1_ref, q1_ref):
    f = jnp.dot(t_ref[...], wt_ref[...], preferred_element_type=jnp.float32)
    f += jnp.dot(d_ref[...], wd_ref[...], preferred_element_type=jnp.float32)
    f += bt_ref[...] + bd_ref[...]
    f_ref[...] = f
    hn0 = _ln(f, g0s[...], g0b[...])
    hn1 = _ln(f, g1s[...], g1b[...])
    z0_ref[...] = jnp.dot(hn0, ws0[...], preferred_element_type=jnp.float32)
    q0_ref[...] = jnp.dot(hn0, wn0[...], preferred_element_type=jnp.float32)
    z1_ref[...] = jnp.dot(hn1, ws1[...], preferred_element_type=jnp.float32)
    q1_ref[...] = jnp.dot(hn1, wn1[...], preferred_element_type=jnp.float32)


@jax.jit
def _pro_lnmm(text_emb, WtT, bt, dw_emb, WdT, bd,
              g0s, g0b, g1s, g1b, ws0, wn0, ws1, wn1):
    vec = pl.BlockSpec((1, D), lambda i: (0, 0))
    mat = pl.BlockSpec((D, D), lambda i: (0, 0))
    blk = pl.BlockSpec((MBLK, D), lambda i: (i, 0))
    return pl.pallas_call(
        _pro_lnmm_body,
        grid=(5,),
        in_specs=[
            pl.BlockSpec((MBLK, DT), lambda i: (i, 0)),
            pl.BlockSpec((DT, D), lambda i: (0, 0)),
            vec, blk, mat, vec,
            vec, vec, vec, vec, mat, mat, mat, mat,
        ],
        out_specs=[blk, blk, blk, blk, blk],
        out_shape=[jax.ShapeDtypeStruct((N, D), jnp.float32)] * 5,
    )(text_emb, WtT, bt, dw_emb, WdT, bd,
      g0s, g0b, g1s, g1b, ws0, wn0, ws1, wn1)


def _comb_lnmm_body(f_ref, z0_ref, s0_ref, d0_ref, z1_ref, s1_ref, d1_ref,
                    b0_ref, b1_ref, g0s, g0b, g1s, g1b, ws0, wn0, ws1, wn1,
                    o_ref, z0o_ref, q0o_ref, z1o_ref, q1o_ref):
    d0 = jnp.maximum(d0_ref[...][:, 0:1], 1.0)
    d1 = jnp.maximum(d1_ref[...][:, 0:1], 1.0)
    y0 = jax.nn.relu(z0_ref[...] + s0_ref[...] / d0 + b0_ref[...])
    y1 = jax.nn.relu(z1_ref[...] + s1_ref[...] / d1 + b1_ref[...])
    f = f_ref[...] + y0 + y1
    o_ref[...] = f
    hn0 = _ln(f, g0s[...], g0b[...])
    hn1 = _ln(f, g1s[...], g1b[...])
    z0o_ref[...] = jnp.dot(hn0, ws0[...], preferred_element_type=jnp.float32)
    q0o_ref[...] = jnp.dot(hn0, wn0[...], preferred_element_type=jnp.float32)
    z1o_ref[...] = jnp.dot(hn1, ws1[...], preferred_element_type=jnp.float32)
    q1o_ref[...] = jnp.dot(hn1, wn1[...], preferred_element_type=jnp.float32)


@jax.jit
def _comb_lnmm(feats, z0, s0, d0, z1, s1, d1, b0, b1,
               g0s, g0b, g1s, g1b, ws0, wn0, ws1, wn1):
    vec = pl.BlockSpec((1, D), lambda i: (0, 0))
    mat = pl.BlockSpec((D, D), lambda i: (0, 0))
    blk = pl.BlockSpec((MBLK, D), lambda i: (i, 0))
    dblk = pl.BlockSpec((MBLK, DEGW), lambda i: (i, 0))
    return pl.pallas_call(
        _comb_lnmm_body,
        grid=(5,),
        in_specs=[blk, blk, blk, dblk, blk, blk, dblk, vec, vec,
                  vec, vec, vec, vec, mat, mat, mat, mat],
        out_specs=[blk, blk, blk, blk, blk],
        out_shape=[jax.ShapeDtypeStruct((N, D), jnp.float32)] * 5,
    )(feats, z0, s0, d0, z1, s1, d1, b0, b1,
      g0s, g0b, g1s, g1b, ws0, wn0, ws1, wn1)


def _combine_body(f_ref, z0_ref, s0_ref, d0_ref, z1_ref, s1_ref, d1_ref,
                  b0_ref, b1_ref, o_ref):
    d0 = jnp.maximum(d0_ref[...][:, 0:1], 1.0)
    d1 = jnp.maximum(d1_ref[...][:, 0:1], 1.0)
    y0 = jax.nn.relu(z0_ref[...] + s0_ref[...] / d0 + b0_ref[...])
    y1 = jax.nn.relu(z1_ref[...] + s1_ref[...] / d1 + b1_ref[...])
    o_ref[...] = f_ref[...] + y0 + y1


@jax.jit
def _combine(feats, z0, s0, d0, z1, s1, d1, b0, b1):
    blk = pl.BlockSpec((MBLK, D), lambda i: (i, 0))
    dblk = pl.BlockSpec((MBLK, DEGW), lambda i: (i, 0))
    vec = pl.BlockSpec((1, D), lambda i: (0, 0))
    return pl.pallas_call(
        _combine_body,
        grid=(5,),
        in_specs=[blk, blk, blk, dblk, blk, blk, dblk, vec, vec],
        out_specs=blk,
        out_shape=jax.ShapeDtypeStruct((N, D), jnp.float32),
    )(feats, z0, s0, d0, z1, s1, d1, b0, b1)


def _norm_body(x_ref, o_ref):
    x = x_ref[...]
    o_ref[...] = x * lax.rsqrt(jnp.sum(x * x, axis=-1, keepdims=True))


@jax.jit
def _normalize(rows):
    return pl.pallas_call(
        _norm_body,
        out_shape=jax.ShapeDtypeStruct((B, D), jnp.float32),
    )(rows)


def kernel(ids, edge_index, nid, text_emb, deepwalk_emb, W_text, b_text,
           W_dw, b_dw, ln_scale, ln_bias, W_self, W_neigh, b_conv):
    e0r = edge_index[0].reshape(E // CH, CH)
    e1r = edge_index[1].reshape(E // CH, CH)
    zeros = jnp.zeros((NPAD, D), jnp.float32)
    zerosd = jnp.zeros((NPAD, DEGW), jnp.float32)
    onesd = jnp.ones((CH, DEGW), jnp.float32)

    def lnp(l):
        return (ln_scale[l, 0].reshape(1, D), ln_bias[l, 0].reshape(1, D),
                ln_scale[l, 1].reshape(1, D), ln_bias[l, 1].reshape(1, D),
                W_self[l, 0].T, W_neigh[l, 0].T,
                W_self[l, 1].T, W_neigh[l, 1].T)

    d0, d1 = _deg(e0r, e1r, zerosd, onesd)
    feats, z0, g0, z1, g1 = _pro_lnmm(
        text_emb, W_text.T, b_text.reshape(1, D),
        deepwalk_emb, W_dw.T, b_dw.reshape(1, D), *lnp(0))
    s0, s1 = _agg(g0, g1, e0r, e1r, zeros)
    feats, z0, g0, z1, g1 = _comb_lnmm(
        feats, z0, s0, d0, z1, s1, d1,
        b_conv[0, 0].reshape(1, D), b_conv[0, 1].reshape(1, D), *lnp(1))
    s0, s1 = _agg(g0, g1, e0r, e1r, zeros)
    feats = _combine(feats, z0, s0, d0, z1, s1, d1,
                     b_conv[1, 0].reshape(1, D), b_conv[1, 1].reshape(1, D))
    rows = _gather(feats, ids)
    return _normalize(rows)
